# baseline (speedup 1.0000x reference)
import jax
import jax.numpy as jnp
from jax import lax
from jax.experimental import pallas as pl
from jax.experimental.pallas import tpu as pltpu
from jax.experimental.pallas import tpu_sc as plsc

N = 10000
E = 320000
D = 128
H = 4
C = 128
HID = 256
NG = 64
NCLS = 10
EN = E + N

NC = 2
NS = 16
NW = NC * NS

KA = 48
NBA = 216
KB = 32
NBB = 324
EPT = NBA * KA
ENP = EPT * NW
NP = 10240
ROWS_PT = NP // NS

_mesh = plsc.VectorSubcoreMesh(core_axis_name="c", subcore_axis_name="s")
_sc_params = pltpu.CompilerParams(use_tc_tiling_on_sc=False,
                                  needs_layout_passes=False)

_DN = lax.GatherDimensionNumbers(offset_dims=(), collapsed_slice_dims=(0,),
                                 start_index_map=(0,))


def _shuf(v, perm):
    return lax.gather(v, perm[:, None], _DN, (1,),
                      mode=lax.GatherScatterMode.PROMISE_IN_BOUNDS)


def _lanesum(v, perms):
    for p in perms:
        v = v + _shuf(v, p)
    return v


def _edge_alpha_body(xl_hbm, xr_hbm, sd_hbm, att_hbm, z16_hbm,
                     ea_out, den_out,
                     att_v, sdv, xlv, xrv, ea_buf, dsc0, dsc1, den_sh,
                     sem_i, sem_r, sem_w0, sem_w1):
    c = lax.axis_index("c")
    s = lax.axis_index("s")
    wid = c * NS + s
    pltpu.sync_copy(att_hbm, att_v)
    pltpu.sync_copy(z16_hbm.at[pl.ds(s * ROWS_PT, ROWS_PT)],
                    den_sh.at[pl.ds(s * ROWS_PT, ROWS_PT)])
    plsc.subcore_barrier()
    lane = lax.iota(jnp.int32, 16)
    perms = [lane ^ sh for sh in (8, 4, 2, 1)]
    att_regs = [att_v[pl.ds(j * 16, 16)] for j in range(32)]

    def idx_start(slot, blk):
        pltpu.async_copy(sd_hbm.at[blk * NW + wid], sdv.at[slot], sem_i)

    def idx_wait(slot):
        pltpu.make_async_copy(sd_hbm.at[wid], sdv.at[slot], sem_i).wait()

    def rows_start(slot):
        pltpu.async_copy(xl_hbm.at[sdv.at[slot, 0]], xlv.at[slot], sem_r)
        pltpu.async_copy(xr_hbm.at[sdv.at[slot, 1]], xrv.at[slot], sem_r)

    def rows_wait(slot):
        pltpu.make_async_copy(xl_hbm.at[sdv.at[slot, 0]], xlv.at[slot],
                              sem_r).wait()
        pltpu.make_async_copy(xr_hbm.at[sdv.at[slot, 1]], xrv.at[slot],
                              sem_r).wait()

    def snap_dst(slot, dsc):
        for i in range(KA // 16):
            dsc[pl.ds(i * 16, 16)] = sdv[slot, 1, pl.ds(i * 16, 16)]

    def compute(slot, blk, dsc, sem_w):
        base = (blk * NW + wid) * KA

        def edge(e, carry):
            sums = []
            for h in range(H):
                acc = jnp.zeros((16,), jnp.float32)
                for jj in range(8):
                    j = h * 8 + jj
                    z = (xlv[slot, e, pl.ds(j * 16, 16)]
                         + xrv[slot, e, pl.ds(j * 16, 16)])
                    acc = acc + jnp.maximum(z, z * 0.2) * att_regs[j]
                sums.append(_lanesum(acc, perms))
            row = jnp.where(lane == 0, sums[0],
                            jnp.where(lane == 1, sums[1],
                                      jnp.where(lane == 2, sums[2],
                                                sums[3])))
            valid = (base + e) < EN
            earow = jnp.where(jnp.logical_and(lane < H, valid),
                              jnp.exp(row), 0.0)
            ea_buf[slot, e, :] = earow
            return carry

        lax.fori_loop(0, KA, edge, 0, unroll=2)
        pltpu.async_copy(ea_buf.at[slot], ea_out.at[pl.ds(base, KA)], sem_w)
        pltpu.sync_copy(ea_buf.at[slot], den_sh.at[dsc], add=True)

    def drain_w(slot, sem_w):
        pltpu.make_async_copy(ea_buf.at[slot], ea_out.at[pl.ds(0, KA)],
                              sem_w).wait()

    idx_start(0, 0)
    idx_wait(0)
    rows_start(0)
    idx_start(1, 1)

    def pair(p, with_wait):
        b0 = 2 * p
        idx_wait(1)
        rows_start(1)
        rows_wait(0)
        if with_wait:
            drain_w(0, sem_w0)
        snap_dst(0, dsc0)
        idx_start(0, jnp.minimum(b0 + 2, NBA - 1))
        compute(0, b0, dsc0, sem_w0)
        idx_wait(0)
        rows_start(0)
        rows_wait(1)
        if with_wait:
            drain_w(1, sem_w1)
        snap_dst(1, dsc1)
        idx_start(1, jnp.minimum(b0 + 3, NBA - 1))
        compute(1, b0 + 1, dsc1, sem_w1)

    pair(0, False)

    def pair_body(p, carry):
        pair(p, True)
        return carry

    lax.fori_loop(1, NBA // 2, pair_body, 0)
    idx_wait(1)
    rows_wait(0)
    drain_w(0, sem_w0)
    drain_w(1, sem_w1)
    plsc.subcore_barrier()
    pltpu.sync_copy(den_sh.at[pl.ds(s * ROWS_PT, ROWS_PT)],
                    den_out.at[c, pl.ds(s * ROWS_PT, ROWS_PT)])


def _edge_scatter_body(xl_hbm, sd_hbm, ea_hbm, invd_hbm, z128_hbm,
                       emb_out,
                       sdv, xlv, ea_v, invd_v, contrib, dsc0, dsc1, emb_sh,
                       sem_i, sem_r, sem_w0, sem_w1):
    c = lax.axis_index("c")
    s = lax.axis_index("s")
    wid = c * NS + s
    pltpu.sync_copy(z128_hbm.at[pl.ds(s * ROWS_PT, ROWS_PT)],
                    emb_sh.at[pl.ds(s * ROWS_PT, ROWS_PT)])
    plsc.subcore_barrier()
    hperms = [jnp.zeros((16,), jnp.int32) + h for h in range(H)]

    def idx_start(slot, blk):
        pltpu.async_copy(sd_hbm.at[blk * NW + wid], sdv.at[slot], sem_i)

    def idx_wait(slot):
        pltpu.make_async_copy(sd_hbm.at[wid], sdv.at[slot], sem_i).wait()

    def rows_start(slot, blk):
        base = (blk * NW + wid) * KB
        pltpu.async_copy(xl_hbm.at[sdv.at[slot, 0]], xlv.at[slot], sem_r)
        pltpu.async_copy(ea_hbm.at[pl.ds(base, KB)], ea_v.at[slot], sem_r)
        pltpu.async_copy(invd_hbm.at[sdv.at[slot, 1]], invd_v.at[slot], sem_r)

    def rows_wait(slot):
        pltpu.make_async_copy(xl_hbm.at[sdv.at[slot, 0]], xlv.at[slot],
                              sem_r).wait()
        pltpu.make_async_copy(ea_hbm.at[pl.ds(0, KB)], ea_v.at[slot],
                              sem_r).wait()
        pltpu.make_async_copy(invd_hbm.at[sdv.at[slot, 1]], invd_v.at[slot],
                              sem_r).wait()

    def snap_dst(slot, dsc):
        for i in range(KB // 16):
            dsc[pl.ds(i * 16, 16)] = sdv[slot, 1, pl.ds(i * 16, 16)]

    def compute(slot, dsc, sem_w):
        def edge(e, carry):
            wrow = ea_v[slot, e, :] * invd_v[slot, e, :]
            ws = [_shuf(wrow, hperms[h]) for h in range(H)]
            for cc in range(8):
                o = xlv[slot, e, pl.ds(cc * 16, 16)] * ws[0]
                for h in range(1, H):
                    o = o + xlv[slot, e, pl.ds(h * 128 + cc * 16, 16)] * ws[h]
                contrib[e, pl.ds(cc * 16, 16)] = o
            return carry

        lax.fori_loop(0, KB, edge, 0, unroll=2)
        pltpu.sync_copy(contrib, emb_sh.at[dsc], add=True)

    def drain_w(slot, sem_w):
        pass

    idx_start(0, 0)
    idx_wait(0)
    rows_start(0, 0)
    idx_start(1, 1)

    def pair(p, with_wait):
        b0 = 2 * p
        idx_wait(1)
        rows_start(1, b0 + 1)
        rows_wait(0)
        if with_wait:
            drain_w(0, sem_w0)
        snap_dst(0, dsc0)
        idx_start(0, jnp.minimum(b0 + 2, NBB - 1))
        compute(0, dsc0, sem_w0)
        idx_wait(0)
        rows_start(0, jnp.minimum(b0 + 2, NBB - 1))
        rows_wait(1)
        if with_wait:
            drain_w(1, sem_w1)
        snap_dst(1, dsc1)
        idx_start(1, jnp.minimum(b0 + 3, NBB - 1))
        compute(1, dsc1, sem_w1)

    pair(0, False)

    def pair_body(p, carry):
        pair(p, True)
        return carry

    lax.fori_loop(1, NBB // 2, pair_body, 0)
    idx_wait(1)
    rows_wait(0)
    drain_w(0, sem_w0)
    drain_w(1, sem_w1)
    plsc.subcore_barrier()
    pltpu.sync_copy(emb_sh.at[pl.ds(s * ROWS_PT, ROWS_PT)],
                    emb_out.at[c, pl.ds(s * ROWS_PT, ROWS_PT)])


def _proj_body(x_ref, wl_ref, wr_ref, bl_ref, br_ref, xl_ref, xr_ref):
    xb = x_ref[...]
    xl_ref[...] = xb @ wl_ref[...] + bl_ref[...]
    xr_ref[...] = xb @ wr_ref[...] + br_ref[...]


def _invd_body(den_ref, out_ref):
    d = den_ref[0] + den_ref[1]
    col = lax.broadcasted_iota(jnp.int32, (NP, 16), 1)
    out_ref[...] = jnp.where(col < H, 0.25 / jnp.maximum(d, 1e-30), 0.0)


def _head_body(emb_ref, bias_ref, batch_ref, w0_ref, w0r_ref, b0_ref,
               b0r_ref, w1_ref, w1r_ref, b1_ref, b1r_ref, fcw_ref, fcb_ref,
               out_ref):
    emb = emb_ref[0] + emb_ref[1] + bias_ref[...]
    bi = batch_ref[...]
    gid = lax.broadcasted_iota(jnp.int32, (NP, NG), 1)
    mask = (bi == gid).astype(jnp.float32)
    sums = lax.dot_general(mask, emb, (((0,), (0,)), ((), ())))
    cnts = jnp.sum(mask, axis=0)[:, None]
    g = sums / jnp.maximum(cnts, 1.0)

    def cell(inp, w, b):
        gates = inp @ w + b
        i, f, gg, o = jnp.split(gates, 4, axis=-1)
        cst = jax.nn.sigmoid(i) * jnp.tanh(gg)
        return jax.nn.sigmoid(o) * jnp.tanh(cst)

    hf = cell(g, w0_ref[...], b0_ref[...])
    hb = cell(g, w0r_ref[...], b0r_ref[...])
    inp1 = jnp.concatenate([hf, hb], axis=-1)
    hf1 = cell(inp1, w1_ref[...], b1_ref[...])
    hb1 = cell(inp1, w1r_ref[...], b1r_ref[...])
    inp2 = jnp.concatenate([hf1, hb1], axis=-1)
    out_ref[...] = inp2 @ fcw_ref[...] + fcb_ref[...]


@jax.jit
def kernel(x, edge_index, batch_index, params):
    p = params
    f32 = jnp.float32

    loops = jnp.arange(N, dtype=jnp.int32)
    src = jnp.pad(jnp.concatenate([edge_index[0].astype(jnp.int32), loops]),
                  (0, ENP - EN))
    dst = jnp.pad(jnp.concatenate([edge_index[1].astype(jnp.int32), loops]),
                  (0, ENP - EN))
    sda = jnp.stack([src.reshape(NW * NBA, KA), dst.reshape(NW * NBA, KA)],
                    axis=1)
    sdb = jnp.stack([src.reshape(NW * NBB, KB), dst.reshape(NW * NBB, KB)],
                    axis=1)
    x_p = jnp.pad(x, ((0, NP - N), (0, 0)))
    att_flat = p['att'].reshape(-1)
    z16 = jnp.zeros((NP, 16), f32)
    z128 = jnp.zeros((NP, 128), f32)
    batch_p = jnp.pad(batch_index.astype(jnp.int32), (0, NP - N),
                      constant_values=-1)[:, None]

    BM = 1024
    xl, xr = pl.pallas_call(
        _proj_body,
        grid=(NP // BM,),
        in_specs=[
            pl.BlockSpec((BM, D), lambda i: (i, 0)),
            pl.BlockSpec((D, H * C), lambda i: (0, 0)),
            pl.BlockSpec((D, H * C), lambda i: (0, 0)),
            pl.BlockSpec((1, H * C), lambda i: (0, 0)),
            pl.BlockSpec((1, H * C), lambda i: (0, 0)),
        ],
        out_specs=[
            pl.BlockSpec((BM, H * C), lambda i: (i, 0)),
            pl.BlockSpec((BM, H * C), lambda i: (i, 0)),
        ],
        out_shape=[
            jax.ShapeDtypeStruct((NP, H * C), f32),
            jax.ShapeDtypeStruct((NP, H * C), f32),
        ],
    )(x_p, p['W_l'], p['W_r'], p['b_l'][None, :], p['b_r'][None, :])

    ea, den = pl.kernel(
        _edge_alpha_body,
        mesh=_mesh,
        compiler_params=_sc_params,
        out_type=[
            jax.ShapeDtypeStruct((ENP, 16), f32),
            jax.ShapeDtypeStruct((NC, NP, 16), f32),
        ],
        scratch_types=[
            pltpu.VMEM((H * C,), f32),
            pltpu.VMEM((2, 2, KA), jnp.int32),
            pltpu.VMEM((2, KA, H * C), f32),
            pltpu.VMEM((2, KA, H * C), f32),
            pltpu.VMEM((2, KA, 16), f32),
            pltpu.VMEM((KA,), jnp.int32),
            pltpu.VMEM((KA,), jnp.int32),
            pltpu.VMEM_SHARED((NP, 16), f32),
            pltpu.SemaphoreType.DMA,
            pltpu.SemaphoreType.DMA,
            pltpu.SemaphoreType.DMA,
            pltpu.SemaphoreType.DMA,
        ],
    )(xl, xr, sda, att_flat, z16)

    invd = pl.pallas_call(
        _invd_body,
        out_shape=jax.ShapeDtypeStruct((NP, 16), f32),
    )(den)

    emb = pl.kernel(
        _edge_scatter_body,
        mesh=_mesh,
        compiler_params=_sc_params,
        out_type=jax.ShapeDtypeStruct((NC, NP, 128), f32),
        scratch_types=[
            pltpu.VMEM((2, 2, KB), jnp.int32),
            pltpu.VMEM((2, KB, H * C), f32),
            pltpu.VMEM((2, KB, 16), f32),
            pltpu.VMEM((2, KB, 16), f32),
            pltpu.VMEM((KB, 128), f32),
            pltpu.VMEM((KB,), jnp.int32),
            pltpu.VMEM((KB,), jnp.int32),
            pltpu.VMEM_SHARED((NP, 128), f32),
            pltpu.SemaphoreType.DMA,
            pltpu.SemaphoreType.DMA,
            pltpu.SemaphoreType.DMA,
            pltpu.SemaphoreType.DMA,
        ],
    )(xl, sdb, ea, invd, z128)

    logits = pl.pallas_call(
        _head_body,
        out_shape=jax.ShapeDtypeStruct((NG, NCLS), f32),
    )(emb, p['gat_bias'][None, :], batch_p,
      p['W_ih_l0'].T, p['W_ih_l0_rev'].T,
      (p['b_ih_l0'] + p['b_hh_l0'])[None, :],
      (p['b_ih_l0_rev'] + p['b_hh_l0_rev'])[None, :],
      p['W_ih_l1'].T, p['W_ih_l1_rev'].T,
      (p['b_ih_l1'] + p['b_hh_l1'])[None, :],
      (p['b_ih_l1_rev'] + p['b_hh_l1_rev'])[None, :],
      p['fc_W'].T, p['fc_b'][None, :])
    return logits


# speedup vs baseline: 1.0670x; 1.0670x over previous
import jax
import jax.numpy as jnp
from jax import lax
from jax.experimental import pallas as pl
from jax.experimental.pallas import tpu as pltpu
from jax.experimental.pallas import tpu_sc as plsc

N = 10000
E = 320000
D = 128
H = 4
C = 128
HID = 256
NG = 64
NCLS = 10
EN = E + N

NC = 2
NS = 16
NW = NC * NS

KA = 48
NBA = 216
KB = 32
NBB = 324
EPT = NBA * KA
ENP = EPT * NW
NP = 10240
ROWS_PT = NP // NS

_mesh = plsc.VectorSubcoreMesh(core_axis_name="c", subcore_axis_name="s")
_sc_params = pltpu.CompilerParams(use_tc_tiling_on_sc=False,
                                  needs_layout_passes=False)

_DN = lax.GatherDimensionNumbers(offset_dims=(), collapsed_slice_dims=(0,),
                                 start_index_map=(0,))


def _shuf(v, perm):
    return lax.gather(v, perm[:, None], _DN, (1,),
                      mode=lax.GatherScatterMode.PROMISE_IN_BOUNDS)


def _lanesum(v, perms):
    for p in perms:
        v = v + _shuf(v, p)
    return v


def _edge_alpha_body(xl_hbm, xr_hbm, sd_hbm, att_hbm, z16_hbm,
                     ea_out, den_out,
                     att_v, sdv, xlv, xrv, ea_buf, dsc0, dsc1, den_sh,
                     sem_i, sem_r, sem_w0, sem_w1):
    c = lax.axis_index("c")
    s = lax.axis_index("s")
    wid = s * NC + c
    pltpu.sync_copy(att_hbm, att_v)
    pltpu.sync_copy(z16_hbm.at[pl.ds(s * ROWS_PT, ROWS_PT)],
                    den_sh.at[pl.ds(s * ROWS_PT, ROWS_PT)])
    plsc.subcore_barrier()
    lane = lax.iota(jnp.int32, 16)
    perms = [lane ^ sh for sh in (8, 4, 2, 1)]
    att_regs = [att_v[pl.ds(j * 16, 16)] for j in range(32)]
    gb0 = wid * NBA
    base0 = wid * EPT

    def idx_start(slot, blk):
        pltpu.async_copy(sd_hbm.at[gb0 + blk], sdv.at[slot], sem_i)

    def idx_wait(slot):
        pltpu.make_async_copy(sd_hbm.at[gb0], sdv.at[slot], sem_i).wait()

    def rows_start(slot):
        pltpu.async_copy(xl_hbm.at[sdv.at[slot, 0]], xlv.at[slot], sem_r)
        pltpu.async_copy(xr_hbm.at[sdv.at[slot, 1]], xrv.at[slot], sem_r)

    def rows_wait(slot):
        pltpu.make_async_copy(xl_hbm.at[sdv.at[slot, 0]], xlv.at[slot],
                              sem_r).wait()
        pltpu.make_async_copy(xr_hbm.at[sdv.at[slot, 1]], xrv.at[slot],
                              sem_r).wait()

    def snap_dst(slot, dsc):
        for i in range(KA // 16):
            dsc[pl.ds(i * 16, 16)] = sdv[slot, 1, pl.ds(i * 16, 16)]

    def compute(slot, blk, dsc, sem_w):
        base = base0 + blk * KA

        def edge(e, carry):
            sums = []
            for h in range(H):
                acc = jnp.zeros((16,), jnp.float32)
                for jj in range(8):
                    j = h * 8 + jj
                    z = (xlv[slot, e, pl.ds(j * 16, 16)]
                         + xrv[slot, e, pl.ds(j * 16, 16)])
                    acc = acc + jnp.maximum(z, z * 0.2) * att_regs[j]
                sums.append(_lanesum(acc, perms))
            row = jnp.where(lane == 0, sums[0],
                            jnp.where(lane == 1, sums[1],
                                      jnp.where(lane == 2, sums[2],
                                                sums[3])))
            valid = (base + e) < EN
            earow = jnp.where(jnp.logical_and(lane < H, valid),
                              jnp.exp(row), 0.0)
            ea_buf[slot, e, :] = earow
            return carry

        lax.fori_loop(0, KA, edge, 0, unroll=2)
        pltpu.async_copy(ea_buf.at[slot], ea_out.at[pl.ds(base, KA)], sem_w)
        pltpu.sync_copy(ea_buf.at[slot], den_sh.at[dsc], add=True)

    def drain_w(slot, sem_w):
        pltpu.make_async_copy(ea_buf.at[slot], ea_out.at[pl.ds(0, KA)],
                              sem_w).wait()

    idx_start(0, 0)
    idx_wait(0)
    rows_start(0)
    idx_start(1, 1)

    def pair(p, with_wait):
        b0 = 2 * p
        idx_wait(1)
        rows_start(1)
        rows_wait(0)
        if with_wait:
            drain_w(0, sem_w0)
        snap_dst(0, dsc0)
        idx_start(0, jnp.minimum(b0 + 2, NBA - 1))
        compute(0, b0, dsc0, sem_w0)
        idx_wait(0)
        rows_start(0)
        rows_wait(1)
        if with_wait:
            drain_w(1, sem_w1)
        snap_dst(1, dsc1)
        idx_start(1, jnp.minimum(b0 + 3, NBA - 1))
        compute(1, b0 + 1, dsc1, sem_w1)

    pair(0, False)

    def pair_body(p, carry):
        pair(p, True)
        return carry

    lax.fori_loop(1, NBA // 2, pair_body, 0)
    idx_wait(1)
    rows_wait(0)
    drain_w(0, sem_w0)
    drain_w(1, sem_w1)
    plsc.subcore_barrier()
    pltpu.sync_copy(den_sh.at[pl.ds(s * ROWS_PT, ROWS_PT)],
                    den_out.at[c, pl.ds(s * ROWS_PT, ROWS_PT)])


def _edge_scatter_body(xl_hbm, sd_hbm, ea_hbm, invd_hbm, z128_hbm,
                       emb_out,
                       sdv, xlv, ea_v, invd_v, contrib, dsc0, dsc1, emb_sh,
                       sem_i, sem_r, sem_w0, sem_w1):
    c = lax.axis_index("c")
    s = lax.axis_index("s")
    wid = s * NC + c
    pltpu.sync_copy(z128_hbm.at[pl.ds(s * ROWS_PT, ROWS_PT)],
                    emb_sh.at[pl.ds(s * ROWS_PT, ROWS_PT)])
    plsc.subcore_barrier()
    hperms = [jnp.zeros((16,), jnp.int32) + h for h in range(H)]
    gb0 = wid * NBB
    base0 = wid * EPT

    def idx_start(slot, blk):
        pltpu.async_copy(sd_hbm.at[gb0 + blk], sdv.at[slot], sem_i)

    def idx_wait(slot):
        pltpu.make_async_copy(sd_hbm.at[gb0], sdv.at[slot], sem_i).wait()

    def rows_start(slot, blk):
        base = base0 + blk * KB
        pltpu.async_copy(xl_hbm.at[sdv.at[slot, 0]], xlv.at[slot], sem_r)
        pltpu.async_copy(ea_hbm.at[pl.ds(base, KB)], ea_v.at[slot], sem_r)
        pltpu.async_copy(invd_hbm.at[sdv.at[slot, 1]], invd_v.at[slot], sem_r)

    def rows_wait(slot):
        pltpu.make_async_copy(xl_hbm.at[sdv.at[slot, 0]], xlv.at[slot],
                              sem_r).wait()
        pltpu.make_async_copy(ea_hbm.at[pl.ds(0, KB)], ea_v.at[slot],
                              sem_r).wait()
        pltpu.make_async_copy(invd_hbm.at[sdv.at[slot, 1]], invd_v.at[slot],
                              sem_r).wait()

    def snap_dst(slot, dsc):
        for i in range(KB // 16):
            dsc[pl.ds(i * 16, 16)] = sdv[slot, 1, pl.ds(i * 16, 16)]

    def compute(slot, dsc, sem_w):
        def edge(e, carry):
            wrow = ea_v[slot, e, :] * invd_v[slot, e, :]
            ws = [_shuf(wrow, hperms[h]) for h in range(H)]
            for cc in range(8):
                o = xlv[slot, e, pl.ds(cc * 16, 16)] * ws[0]
                for h in range(1, H):
                    o = o + xlv[slot, e, pl.ds(h * 128 + cc * 16, 16)] * ws[h]
                contrib[e, pl.ds(cc * 16, 16)] = o
            return carry

        lax.fori_loop(0, KB, edge, 0, unroll=2)
        pltpu.sync_copy(contrib, emb_sh.at[dsc], add=True)

    def drain_w(slot, sem_w):
        pass

    idx_start(0, 0)
    idx_wait(0)
    rows_start(0, 0)
    idx_start(1, 1)

    def pair(p, with_wait):
        b0 = 2 * p
        idx_wait(1)
        rows_start(1, b0 + 1)
        rows_wait(0)
        if with_wait:
            drain_w(0, sem_w0)
        snap_dst(0, dsc0)
        idx_start(0, jnp.minimum(b0 + 2, NBB - 1))
        compute(0, dsc0, sem_w0)
        idx_wait(0)
        rows_start(0, jnp.minimum(b0 + 2, NBB - 1))
        rows_wait(1)
        if with_wait:
            drain_w(1, sem_w1)
        snap_dst(1, dsc1)
        idx_start(1, jnp.minimum(b0 + 3, NBB - 1))
        compute(1, dsc1, sem_w1)

    pair(0, False)

    def pair_body(p, carry):
        pair(p, True)
        return carry

    lax.fori_loop(1, NBB // 2, pair_body, 0)
    idx_wait(1)
    rows_wait(0)
    drain_w(0, sem_w0)
    drain_w(1, sem_w1)
    plsc.subcore_barrier()
    pltpu.sync_copy(emb_sh.at[pl.ds(s * ROWS_PT, ROWS_PT)],
                    emb_out.at[c, pl.ds(s * ROWS_PT, ROWS_PT)])


def _proj_body(x_ref, wl_ref, wr_ref, bl_ref, br_ref, xl_ref, xr_ref):
    xb = x_ref[...]
    xl_ref[...] = xb @ wl_ref[...] + bl_ref[...]
    xr_ref[...] = xb @ wr_ref[...] + br_ref[...]


def _invd_body(den_ref, out_ref):
    d = den_ref[0] + den_ref[1]
    col = lax.broadcasted_iota(jnp.int32, (NP, 16), 1)
    out_ref[...] = jnp.where(col < H, 0.25 / jnp.maximum(d, 1e-30), 0.0)


def _head_body(emb_ref, bias_ref, batch_ref, w0_ref, w0r_ref, b0_ref,
               b0r_ref, w1_ref, w1r_ref, b1_ref, b1r_ref, fcw_ref, fcb_ref,
               out_ref):
    emb = emb_ref[0] + emb_ref[1] + bias_ref[...]
    bi = batch_ref[...]
    gid = lax.broadcasted_iota(jnp.int32, (NP, NG), 1)
    mask = (bi == gid).astype(jnp.float32)
    sums = lax.dot_general(mask, emb, (((0,), (0,)), ((), ())))
    cnts = jnp.sum(mask, axis=0)[:, None]
    g = sums / jnp.maximum(cnts, 1.0)

    def cell(inp, w, b):
        gates = inp @ w + b
        i, f, gg, o = jnp.split(gates, 4, axis=-1)
        cst = jax.nn.sigmoid(i) * jnp.tanh(gg)
        return jax.nn.sigmoid(o) * jnp.tanh(cst)

    hf = cell(g, w0_ref[...], b0_ref[...])
    hb = cell(g, w0r_ref[...], b0r_ref[...])
    inp1 = jnp.concatenate([hf, hb], axis=-1)
    hf1 = cell(inp1, w1_ref[...], b1_ref[...])
    hb1 = cell(inp1, w1r_ref[...], b1r_ref[...])
    inp2 = jnp.concatenate([hf1, hb1], axis=-1)
    out_ref[...] = inp2 @ fcw_ref[...] + fcb_ref[...]


@jax.jit
def kernel(x, edge_index, batch_index, params):
    p = params
    f32 = jnp.float32

    loops = jnp.arange(N, dtype=jnp.int32)
    src = jnp.pad(jnp.concatenate([edge_index[0].astype(jnp.int32), loops]),
                  (0, ENP - EN))
    dst = jnp.pad(jnp.concatenate([edge_index[1].astype(jnp.int32), loops]),
                  (0, ENP - EN))
    sda = jnp.stack([src.reshape(NW * NBA, KA), dst.reshape(NW * NBA, KA)],
                    axis=1)
    sdb = jnp.stack([src.reshape(NW * NBB, KB), dst.reshape(NW * NBB, KB)],
                    axis=1)
    x_p = jnp.pad(x, ((0, NP - N), (0, 0)))
    att_flat = p['att'].reshape(-1)
    z16 = jnp.zeros((NP, 16), f32)
    z128 = jnp.zeros((NP, 128), f32)
    batch_p = jnp.pad(batch_index.astype(jnp.int32), (0, NP - N),
                      constant_values=-1)[:, None]

    BM = 1024
    xl, xr = pl.pallas_call(
        _proj_body,
        grid=(NP // BM,),
        in_specs=[
            pl.BlockSpec((BM, D), lambda i: (i, 0)),
            pl.BlockSpec((D, H * C), lambda i: (0, 0)),
            pl.BlockSpec((D, H * C), lambda i: (0, 0)),
            pl.BlockSpec((1, H * C), lambda i: (0, 0)),
            pl.BlockSpec((1, H * C), lambda i: (0, 0)),
        ],
        out_specs=[
            pl.BlockSpec((BM, H * C), lambda i: (i, 0)),
            pl.BlockSpec((BM, H * C), lambda i: (i, 0)),
        ],
        out_shape=[
            jax.ShapeDtypeStruct((NP, H * C), f32),
            jax.ShapeDtypeStruct((NP, H * C), f32),
        ],
    )(x_p, p['W_l'], p['W_r'], p['b_l'][None, :], p['b_r'][None, :])

    ea, den = pl.kernel(
        _edge_alpha_body,
        mesh=_mesh,
        compiler_params=_sc_params,
        out_type=[
            jax.ShapeDtypeStruct((ENP, 16), f32),
            jax.ShapeDtypeStruct((NC, NP, 16), f32),
        ],
        scratch_types=[
            pltpu.VMEM((H * C,), f32),
            pltpu.VMEM((2, 2, KA), jnp.int32),
            pltpu.VMEM((2, KA, H * C), f32),
            pltpu.VMEM((2, KA, H * C), f32),
            pltpu.VMEM((2, KA, 16), f32),
            pltpu.VMEM((KA,), jnp.int32),
            pltpu.VMEM((KA,), jnp.int32),
            pltpu.VMEM_SHARED((NP, 16), f32),
            pltpu.SemaphoreType.DMA,
            pltpu.SemaphoreType.DMA,
            pltpu.SemaphoreType.DMA,
            pltpu.SemaphoreType.DMA,
        ],
    )(xl, xr, sda, att_flat, z16)

    invd = pl.pallas_call(
        _invd_body,
        out_shape=jax.ShapeDtypeStruct((NP, 16), f32),
    )(den)

    emb = pl.kernel(
        _edge_scatter_body,
        mesh=_mesh,
        compiler_params=_sc_params,
        out_type=jax.ShapeDtypeStruct((NC, NP, 128), f32),
        scratch_types=[
            pltpu.VMEM((2, 2, KB), jnp.int32),
            pltpu.VMEM((2, KB, H * C), f32),
            pltpu.VMEM((2, KB, 16), f32),
            pltpu.VMEM((2, KB, 16), f32),
            pltpu.VMEM((KB, 128), f32),
            pltpu.VMEM((KB,), jnp.int32),
            pltpu.VMEM((KB,), jnp.int32),
            pltpu.VMEM_SHARED((NP, 128), f32),
            pltpu.SemaphoreType.DMA,
            pltpu.SemaphoreType.DMA,
            pltpu.SemaphoreType.DMA,
            pltpu.SemaphoreType.DMA,
        ],
    )(xl, sdb, ea, invd, z128)

    logits = pl.pallas_call(
        _head_body,
        out_shape=jax.ShapeDtypeStruct((NG, NCLS), f32),
    )(emb, p['gat_bias'][None, :], batch_p,
      p['W_ih_l0'].T, p['W_ih_l0_rev'].T,
      (p['b_ih_l0'] + p['b_hh_l0'])[None, :],
      (p['b_ih_l0_rev'] + p['b_hh_l0_rev'])[None, :],
      p['W_ih_l1'].T, p['W_ih_l1_rev'].T,
      (p['b_ih_l1'] + p['b_hh_l1'])[None, :],
      (p['b_ih_l1_rev'] + p['b_hh_l1_rev'])[None, :],
      p['fc_W'].T, p['fc_b'][None, :])
    return logits


# speedup vs baseline: 1.1130x; 1.0432x over previous
import jax
import jax.numpy as jnp
from jax import lax
from jax.experimental import pallas as pl
from jax.experimental.pallas import tpu as pltpu
from jax.experimental.pallas import tpu_sc as plsc

N = 10000
E = 320000
D = 128
H = 4
C = 128
HID = 256
NG = 64
NCLS = 10
EN = E + N

NC = 2
NS = 16
NW = NC * NS

KA = 48
NBA = 216
KB = 32
NBB = 324
EPT = NBA * KA
ENP = EPT * NW
NP = 10240
ROWS_PT = NP // NS

_mesh = plsc.VectorSubcoreMesh(core_axis_name="c", subcore_axis_name="s")
_sc_params = pltpu.CompilerParams(use_tc_tiling_on_sc=False,
                                  needs_layout_passes=False)

_DN = lax.GatherDimensionNumbers(offset_dims=(), collapsed_slice_dims=(0,),
                                 start_index_map=(0,))


def _shuf(v, perm):
    return lax.gather(v, perm[:, None], _DN, (1,),
                      mode=lax.GatherScatterMode.PROMISE_IN_BOUNDS)


def _lanesum(v, perms):
    for p in perms:
        v = v + _shuf(v, p)
    return v


def _edge_alpha_body(xl_hbm, xr_hbm, sd_hbm, att_hbm, z16_hbm,
                     ea_out, den_out,
                     att_v, sdv, xlv, xrv, ea_buf, dsc0, dsc1, den_sh,
                     sem_i, sem_r, sem_w0, sem_w1, sem_a0, sem_a1):
    c = lax.axis_index("c")
    s = lax.axis_index("s")
    wid = s * NC + c
    pltpu.sync_copy(att_hbm, att_v)
    pltpu.sync_copy(z16_hbm.at[pl.ds(s * ROWS_PT, ROWS_PT)],
                    den_sh.at[pl.ds(s * ROWS_PT, ROWS_PT)])
    plsc.subcore_barrier()
    lane = lax.iota(jnp.int32, 16)
    perms = [lane ^ sh for sh in (8, 4, 2, 1)]
    att_regs = [att_v[pl.ds(j * 16, 16)] for j in range(32)]
    gb0 = wid * NBA
    base0 = wid * EPT

    def idx_start(slot, blk):
        pltpu.async_copy(sd_hbm.at[gb0 + blk], sdv.at[slot], sem_i)

    def idx_wait(slot):
        pltpu.make_async_copy(sd_hbm.at[gb0], sdv.at[slot], sem_i).wait()

    def rows_start(slot):
        pltpu.async_copy(xl_hbm.at[sdv.at[slot, 0]], xlv.at[slot], sem_r)
        pltpu.async_copy(xr_hbm.at[sdv.at[slot, 1]], xrv.at[slot], sem_r)

    def rows_wait(slot):
        pltpu.make_async_copy(xl_hbm.at[sdv.at[slot, 0]], xlv.at[slot],
                              sem_r).wait()
        pltpu.make_async_copy(xr_hbm.at[sdv.at[slot, 1]], xrv.at[slot],
                              sem_r).wait()

    def snap_dst(slot, dsc):
        for i in range(KA // 16):
            dsc[pl.ds(i * 16, 16)] = sdv[slot, 1, pl.ds(i * 16, 16)]

    def compute(slot, blk, dsc, sem_w, sem_a):
        base = base0 + blk * KA

        def edge(e, carry):
            sums = []
            for h in range(H):
                acc = jnp.zeros((16,), jnp.float32)
                for jj in range(8):
                    j = h * 8 + jj
                    z = (xlv[slot, e, pl.ds(j * 16, 16)]
                         + xrv[slot, e, pl.ds(j * 16, 16)])
                    acc = acc + jnp.maximum(z, z * 0.2) * att_regs[j]
                sums.append(_lanesum(acc, perms))
            row = jnp.where(lane == 0, sums[0],
                            jnp.where(lane == 1, sums[1],
                                      jnp.where(lane == 2, sums[2],
                                                sums[3])))
            valid = (base + e) < EN
            earow = jnp.where(jnp.logical_and(lane < H, valid),
                              jnp.exp(row), 0.0)
            ea_buf[slot, e, :] = earow
            return carry

        lax.fori_loop(0, KA, edge, 0, unroll=2)
        pltpu.async_copy(ea_buf.at[slot], ea_out.at[pl.ds(base, KA)], sem_w)
        pltpu.async_copy(ea_buf.at[slot], den_sh.at[dsc], sem_a, add=True)

    def drain_w(slot, sem_w):
        pltpu.make_async_copy(ea_buf.at[slot], ea_out.at[pl.ds(0, KA)],
                              sem_w).wait()

    def drain_a(slot, dsc, sem_a):
        pltpu.make_async_copy(ea_buf.at[slot], den_sh.at[dsc], sem_a).wait()

    idx_start(0, 0)
    idx_wait(0)
    rows_start(0)
    idx_start(1, 1)

    def pair(p, with_wait):
        b0 = 2 * p
        idx_wait(1)
        rows_start(1)
        rows_wait(0)
        if with_wait:
            drain_w(0, sem_w0)
            drain_a(0, dsc0, sem_a0)
        snap_dst(0, dsc0)
        idx_start(0, jnp.minimum(b0 + 2, NBA - 1))
        compute(0, b0, dsc0, sem_w0, sem_a0)
        idx_wait(0)
        rows_start(0)
        rows_wait(1)
        if with_wait:
            drain_w(1, sem_w1)
            drain_a(1, dsc1, sem_a1)
        snap_dst(1, dsc1)
        idx_start(1, jnp.minimum(b0 + 3, NBA - 1))
        compute(1, b0 + 1, dsc1, sem_w1, sem_a1)

    pair(0, False)

    def pair_body(p, carry):
        pair(p, True)
        return carry

    lax.fori_loop(1, NBA // 2, pair_body, 0)
    idx_wait(1)
    rows_wait(0)
    drain_w(0, sem_w0)
    drain_w(1, sem_w1)
    drain_a(0, dsc0, sem_a0)
    drain_a(1, dsc1, sem_a1)
    plsc.subcore_barrier()
    pltpu.sync_copy(den_sh.at[pl.ds(s * ROWS_PT, ROWS_PT)],
                    den_out.at[c, pl.ds(s * ROWS_PT, ROWS_PT)])


def _edge_scatter_body(xl_hbm, sd_hbm, ea_hbm, invd_hbm, z128_hbm,
                       emb_out,
                       sdv, xlv, ea_v, invd_v, contrib, dsc0, dsc1, emb_sh,
                       sem_i, sem_r, sem_w0, sem_w1):
    c = lax.axis_index("c")
    s = lax.axis_index("s")
    wid = s * NC + c
    pltpu.sync_copy(z128_hbm.at[pl.ds(s * ROWS_PT, ROWS_PT)],
                    emb_sh.at[pl.ds(s * ROWS_PT, ROWS_PT)])
    plsc.subcore_barrier()
    hperms = [jnp.zeros((16,), jnp.int32) + h for h in range(H)]
    gb0 = wid * NBB
    base0 = wid * EPT

    def idx_start(slot, blk):
        pltpu.async_copy(sd_hbm.at[gb0 + blk], sdv.at[slot], sem_i)

    def idx_wait(slot):
        pltpu.make_async_copy(sd_hbm.at[gb0], sdv.at[slot], sem_i).wait()

    def rows_start(slot, blk):
        base = base0 + blk * KB
        pltpu.async_copy(xl_hbm.at[sdv.at[slot, 0]], xlv.at[slot], sem_r)
        pltpu.async_copy(ea_hbm.at[pl.ds(base, KB)], ea_v.at[slot], sem_r)
        pltpu.async_copy(invd_hbm.at[sdv.at[slot, 1]], invd_v.at[slot], sem_r)

    def rows_wait(slot):
        pltpu.make_async_copy(xl_hbm.at[sdv.at[slot, 0]], xlv.at[slot],
                              sem_r).wait()
        pltpu.make_async_copy(ea_hbm.at[pl.ds(0, KB)], ea_v.at[slot],
                              sem_r).wait()
        pltpu.make_async_copy(invd_hbm.at[sdv.at[slot, 1]], invd_v.at[slot],
                              sem_r).wait()

    def snap_dst(slot, dsc):
        for i in range(KB // 16):
            dsc[pl.ds(i * 16, 16)] = sdv[slot, 1, pl.ds(i * 16, 16)]

    def compute(slot, dsc, sem_w):
        def edge(e, carry):
            wrow = ea_v[slot, e, :] * invd_v[slot, e, :]
            ws = [_shuf(wrow, hperms[h]) for h in range(H)]
            for cc in range(8):
                o = xlv[slot, e, pl.ds(cc * 16, 16)] * ws[0]
                for h in range(1, H):
                    o = o + xlv[slot, e, pl.ds(h * 128 + cc * 16, 16)] * ws[h]
                contrib[slot, e, pl.ds(cc * 16, 16)] = o
            return carry

        lax.fori_loop(0, KB, edge, 0, unroll=2)
        pltpu.async_copy(contrib.at[slot], emb_sh.at[dsc], sem_w, add=True)

    def drain_w(slot, dsc, sem_w):
        pltpu.make_async_copy(contrib.at[slot], emb_sh.at[dsc], sem_w).wait()

    idx_start(0, 0)
    idx_wait(0)
    rows_start(0, 0)
    idx_start(1, 1)

    def pair(p, with_wait):
        b0 = 2 * p
        idx_wait(1)
        rows_start(1, b0 + 1)
        rows_wait(0)
        if with_wait:
            drain_w(0, dsc0, sem_w0)
        snap_dst(0, dsc0)
        idx_start(0, jnp.minimum(b0 + 2, NBB - 1))
        compute(0, dsc0, sem_w0)
        idx_wait(0)
        rows_start(0, jnp.minimum(b0 + 2, NBB - 1))
        rows_wait(1)
        if with_wait:
            drain_w(1, dsc1, sem_w1)
        snap_dst(1, dsc1)
        idx_start(1, jnp.minimum(b0 + 3, NBB - 1))
        compute(1, dsc1, sem_w1)

    pair(0, False)

    def pair_body(p, carry):
        pair(p, True)
        return carry

    lax.fori_loop(1, NBB // 2, pair_body, 0)
    idx_wait(1)
    rows_wait(0)
    drain_w(0, dsc0, sem_w0)
    drain_w(1, dsc1, sem_w1)
    plsc.subcore_barrier()
    pltpu.sync_copy(emb_sh.at[pl.ds(s * ROWS_PT, ROWS_PT)],
                    emb_out.at[c, pl.ds(s * ROWS_PT, ROWS_PT)])


def _proj_body(x_ref, wl_ref, wr_ref, bl_ref, br_ref, xl_ref, xr_ref):
    xb = x_ref[...]
    xl_ref[...] = xb @ wl_ref[...] + bl_ref[...]
    xr_ref[...] = xb @ wr_ref[...] + br_ref[...]


def _invd_body(den_ref, out_ref):
    d = den_ref[0] + den_ref[1]
    col = lax.broadcasted_iota(jnp.int32, (NP, 16), 1)
    out_ref[...] = jnp.where(col < H, 0.25 / jnp.maximum(d, 1e-30), 0.0)


def _head_body(emb_ref, bias_ref, batch_ref, w0_ref, w0r_ref, b0_ref,
               b0r_ref, w1_ref, w1r_ref, b1_ref, b1r_ref, fcw_ref, fcb_ref,
               out_ref):
    emb = emb_ref[0] + emb_ref[1] + bias_ref[...]
    bi = batch_ref[...]
    gid = lax.broadcasted_iota(jnp.int32, (NP, NG), 1)
    mask = (bi == gid).astype(jnp.float32)
    sums = lax.dot_general(mask, emb, (((0,), (0,)), ((), ())))
    cnts = jnp.sum(mask, axis=0)[:, None]
    g = sums / jnp.maximum(cnts, 1.0)

    def cell(inp, w, b):
        gates = inp @ w + b
        i, f, gg, o = jnp.split(gates, 4, axis=-1)
        cst = jax.nn.sigmoid(i) * jnp.tanh(gg)
        return jax.nn.sigmoid(o) * jnp.tanh(cst)

    hf = cell(g, w0_ref[...], b0_ref[...])
    hb = cell(g, w0r_ref[...], b0r_ref[...])
    inp1 = jnp.concatenate([hf, hb], axis=-1)
    hf1 = cell(inp1, w1_ref[...], b1_ref[...])
    hb1 = cell(inp1, w1r_ref[...], b1r_ref[...])
    inp2 = jnp.concatenate([hf1, hb1], axis=-1)
    out_ref[...] = inp2 @ fcw_ref[...] + fcb_ref[...]


@jax.jit
def kernel(x, edge_index, batch_index, params):
    p = params
    f32 = jnp.float32

    loops = jnp.arange(N, dtype=jnp.int32)
    src = jnp.pad(jnp.concatenate([edge_index[0].astype(jnp.int32), loops]),
                  (0, ENP - EN))
    dst = jnp.pad(jnp.concatenate([edge_index[1].astype(jnp.int32), loops]),
                  (0, ENP - EN))
    sda = jnp.stack([src.reshape(NW * NBA, KA), dst.reshape(NW * NBA, KA)],
                    axis=1)
    sdb = jnp.stack([src.reshape(NW * NBB, KB), dst.reshape(NW * NBB, KB)],
                    axis=1)
    x_p = jnp.pad(x, ((0, NP - N), (0, 0)))
    att_flat = p['att'].reshape(-1)
    z16 = jnp.zeros((NP, 16), f32)
    z128 = jnp.zeros((NP, 128), f32)
    batch_p = jnp.pad(batch_index.astype(jnp.int32), (0, NP - N),
                      constant_values=-1)[:, None]

    BM = 1024
    xl, xr = pl.pallas_call(
        _proj_body,
        grid=(NP // BM,),
        in_specs=[
            pl.BlockSpec((BM, D), lambda i: (i, 0)),
            pl.BlockSpec((D, H * C), lambda i: (0, 0)),
            pl.BlockSpec((D, H * C), lambda i: (0, 0)),
            pl.BlockSpec((1, H * C), lambda i: (0, 0)),
            pl.BlockSpec((1, H * C), lambda i: (0, 0)),
        ],
        out_specs=[
            pl.BlockSpec((BM, H * C), lambda i: (i, 0)),
            pl.BlockSpec((BM, H * C), lambda i: (i, 0)),
        ],
        out_shape=[
            jax.ShapeDtypeStruct((NP, H * C), f32),
            jax.ShapeDtypeStruct((NP, H * C), f32),
        ],
    )(x_p, p['W_l'], p['W_r'], p['b_l'][None, :], p['b_r'][None, :])

    ea, den = pl.kernel(
        _edge_alpha_body,
        mesh=_mesh,
        compiler_params=_sc_params,
        out_type=[
            jax.ShapeDtypeStruct((ENP, 16), f32),
            jax.ShapeDtypeStruct((NC, NP, 16), f32),
        ],
        scratch_types=[
            pltpu.VMEM((H * C,), f32),
            pltpu.VMEM((2, 2, KA), jnp.int32),
            pltpu.VMEM((2, KA, H * C), f32),
            pltpu.VMEM((2, KA, H * C), f32),
            pltpu.VMEM((2, KA, 16), f32),
            pltpu.VMEM((KA,), jnp.int32),
            pltpu.VMEM((KA,), jnp.int32),
            pltpu.VMEM_SHARED((NP, 16), f32),
            pltpu.SemaphoreType.DMA,
            pltpu.SemaphoreType.DMA,
            pltpu.SemaphoreType.DMA,
            pltpu.SemaphoreType.DMA,
            pltpu.SemaphoreType.DMA,
            pltpu.SemaphoreType.DMA,
        ],
    )(xl, xr, sda, att_flat, z16)

    invd = pl.pallas_call(
        _invd_body,
        out_shape=jax.ShapeDtypeStruct((NP, 16), f32),
    )(den)

    emb = pl.kernel(
        _edge_scatter_body,
        mesh=_mesh,
        compiler_params=_sc_params,
        out_type=jax.ShapeDtypeStruct((NC, NP, 128), f32),
        scratch_types=[
            pltpu.VMEM((2, 2, KB), jnp.int32),
            pltpu.VMEM((2, KB, H * C), f32),
            pltpu.VMEM((2, KB, 16), f32),
            pltpu.VMEM((2, KB, 16), f32),
            pltpu.VMEM((2, KB, 128), f32),
            pltpu.VMEM((KB,), jnp.int32),
            pltpu.VMEM((KB,), jnp.int32),
            pltpu.VMEM_SHARED((NP, 128), f32),
            pltpu.SemaphoreType.DMA,
            pltpu.SemaphoreType.DMA,
            pltpu.SemaphoreType.DMA,
            pltpu.SemaphoreType.DMA,
        ],
    )(xl, sdb, ea, invd, z128)

    logits = pl.pallas_call(
        _head_body,
        out_shape=jax.ShapeDtypeStruct((NG, NCLS), f32),
    )(emb, p['gat_bias'][None, :], batch_p,
      p['W_ih_l0'].T, p['W_ih_l0_rev'].T,
      (p['b_ih_l0'] + p['b_hh_l0'])[None, :],
      (p['b_ih_l0_rev'] + p['b_hh_l0_rev'])[None, :],
      p['W_ih_l1'].T, p['W_ih_l1_rev'].T,
      (p['b_ih_l1'] + p['b_hh_l1'])[None, :],
      (p['b_ih_l1_rev'] + p['b_hh_l1_rev'])[None, :],
      p['fc_W'].T, p['fc_b'][None, :])
    return logits
